# baseline (speedup 1.0000x reference)
import jax
import jax.numpy as jnp
import numpy as np
from jax import lax
from jax.experimental import pallas as pl
from jax.experimental.pallas import tpu as pltpu
from jax.experimental.pallas import tpu_sc as plsc

_NC = 2
_NS = 16

_QQ = 196
_KK = 196
_H = 12
_QBLK = 98
_NQC = _QQ // _QBLK
_PLANE = _KK * _QBLK
_PPAD = 19216
_PVEC = _PPAD // 16
_TAB = _QQ * _H


def _bb_pos_planes() -> np.ndarray:
    q_blocks_h = int(np.sqrt(float(_QQ)))
    k_blocks_h = int(np.sqrt(float(_KK)))
    strides = int(np.ceil(np.sqrt(float(_KK) / float(_QQ))))
    x1, y1 = np.meshgrid(np.arange(q_blocks_h), np.arange(q_blocks_h))
    x2, y2 = np.meshgrid(np.arange(k_blocks_h), np.arange(k_blocks_h))
    aa = np.stack([x1.reshape(-1), y1.reshape(-1)], axis=-1)
    bb_grid = np.stack([x2.reshape(-1), y2.reshape(-1)], axis=-1)
    diff = np.abs(bb_grid[None, :, :] - aa[:, None, :] * strides)
    bb_pos = (diff[:, :, 0] + diff[:, :, 1] * k_blocks_h).astype(np.int64)

    p = np.arange(_PLANE, dtype=np.int64)
    k = p // _QBLK
    qp = p % _QBLK
    planes = np.zeros((_NQC, _PPAD), dtype=np.int32)
    for qc in range(_NQC):
        planes[qc, :_PLANE] = bb_pos[qc * _QBLK + qp, k]
    return planes


_IDX_NP = _bb_pos_planes()


def _sc_gather_body(bb_hbm, idx_hbm, out_hbm, table_v, idx_v, plane_v):
    wid = lax.axis_index("s") * _NC + lax.axis_index("c")

    @pl.when(wid < _H * _NQC)
    def _():
        h = wid // _NQC
        qc = wid % _NQC
        pltpu.sync_copy(bb_hbm, table_v)
        pltpu.sync_copy(idx_hbm.at[qc], idx_v)
        hoff = h * _QQ

        @plsc.parallel_loop(0, _PVEC - 1, unroll=16)
        def body(i):
            sl = pl.ds(i * 16, 16)
            p = i * 16 + lax.iota(jnp.int32, 16)
            vals = plsc.load_gather(table_v, [idx_v[sl] + hoff])
            plsc.store_scatter(plane_v, [p // _QBLK, p % _QBLK], vals)

        last = _PVEC - 1
        p = last * 16 + lax.iota(jnp.int32, 16)
        vals = plsc.load_gather(table_v, [idx_v[pl.ds(last * 16, 16)] + hoff])
        plsc.store_scatter(
            plane_v, [p // _QBLK, p % _QBLK], vals, mask=p < _PLANE
        )

        pltpu.sync_copy(plane_v, out_hbm.at[h, qc])


def _sc_gather(bb_t_flat, idx):
    mesh = plsc.VectorSubcoreMesh(core_axis_name="c", subcore_axis_name="s")
    fn = pl.kernel(
        _sc_gather_body,
        out_type=jax.ShapeDtypeStruct((_H, _NQC, _KK, _QBLK), jnp.float32),
        mesh=mesh,
        scratch_types=[
            pltpu.VMEM((_TAB,), jnp.float32),
            pltpu.VMEM((_PPAD,), jnp.int32),
            pltpu.VMEM((_KK, _QBLK), jnp.float32),
        ],
        compiler_params=pltpu.CompilerParams(needs_layout_passes=False),
    )
    return fn(bb_t_flat, idx)


def _add_body(x_ref, b_ref, o_ref):
    for q in range(_QBLK):
        o_ref[0, q] = x_ref[0, q] + b_ref[0, 0, :, q : q + 1]


def _tc_add(x_t, bias_t, n_batch):
    return pl.pallas_call(
        _add_body,
        grid=(_H, _NQC),
        in_specs=[
            pl.BlockSpec((1, _QBLK, _KK, n_batch), lambda h, qc: (h, qc, 0, 0)),
            pl.BlockSpec((1, 1, _KK, _QBLK), lambda h, qc: (h, qc, 0, 0)),
        ],
        out_specs=pl.BlockSpec((1, _QBLK, _KK, n_batch), lambda h, qc: (h, qc, 0, 0)),
        out_shape=jax.ShapeDtypeStruct(x_t.shape, x_t.dtype),
    )(x_t, bias_t)


def kernel(inputs, bb):
    n_batch = inputs.shape[0]
    bb_t_flat = jnp.transpose(bb, (1, 0)).reshape(-1)
    idx = jnp.asarray(_IDX_NP)
    bias_t = _sc_gather(bb_t_flat, idx)
    x_t = jnp.transpose(inputs, (1, 2, 3, 0))
    out_t = _tc_add(x_t, bias_t, n_batch)
    return jnp.transpose(out_t, (3, 0, 1, 2))
